# baseline (speedup 1.0000x reference)
import jax
import jax.numpy as jnp
from jax import lax
from jax.experimental import pallas as pl
from jax.experimental.pallas import tpu as pltpu
from jax.experimental.pallas import tpu_sc as plsc

NUM_FIELDS = 26
NUM_BUCKETS = 100000
EMBSIZE = 32
BATCH = 16384

_INFO = plsc.get_sparse_core_info()
NC, NS, NL = _INFO.num_cores, _INFO.num_subcores, _INFO.num_lanes
NW = NC * NS
NROWS = NUM_FIELDS * EMBSIZE
RPW = NROWS // NW
NVEC = BATCH // NL

SLOT_W = 25088
NSLOT_CH = 3
TAIL_B0 = NSLOT_CH * SLOT_W
TAIL_W = NUM_BUCKETS - TAIL_B0
NCH = NSLOT_CH + 1


def _body(tab_hbm, idx_hbm, out_hbm, chk_v, tail_v, idx_v, ob_v, gsem, wsem):
    wid = lax.axis_index("s") * NC + lax.axis_index("c")
    r0 = wid * RPW

    def chunk_copy(r, c, k):
        f = r // EMBSIZE
        e = r % EMBSIZE
        if c == NSLOT_CH:
            dst = tail_v
            b0, w = TAIL_B0, TAIL_W
        else:
            dst = chk_v.at[lax.rem(k + c, 2)]
            b0, w = c * SLOT_W, SLOT_W
        return pltpu.make_async_copy(
            tab_hbm.at[f, pl.ds(e, 1), pl.ds(b0, w)], dst, gsem
        )

    def row_step(k, f_prev):
        r = r0 + k
        f = r // EMBSIZE
        ob_s = lax.rem(k, 2)

        @pl.when(jnp.logical_or(k == 0, f != f_prev))
        def _():
            pltpu.sync_copy(idx_hbm.at[pl.ds(f, 1)], idx_v)

        @pl.when(k >= 2)
        def _():
            pltpu.make_async_copy(
                ob_v.at[ob_s], out_hbm.at[pl.ds(r - 2, 1)], wsem
            ).wait()

        zero16 = jnp.zeros((NL,), jnp.int32)
        lane16 = lax.iota(jnp.int32, NL)
        swu = jnp.uint32(SLOT_W)
        swm1 = jnp.uint32(SLOT_W - 1)
        twu = jnp.uint32(TAIL_W)
        twm1 = jnp.uint32(TAIL_W - 1)

        chunk_copy(r, 1, k).start()
        chunk_copy(r, NSLOT_CH, k).start()
        chunk_copy(r, 0, k).wait()

        src0 = chk_v.at[lax.rem(k, 2)]

        @plsc.parallel_loop(0, NVEC, 1, unroll=2)
        def p0(i):
            idx16 = idx_v[0, pl.ds(i * NL, NL)]
            lu = plsc.bitcast(idx16, jnp.uint32)
            loc = plsc.bitcast(jnp.minimum(lu, swm1), jnp.int32)
            g = plsc.load_gather(src0, [zero16, loc])
            ob_v.at[ob_s, 0][pl.ds(i * NL, NL)] = g

        chunk_copy(r, 2, k).start()
        chunk_copy(r, 1, k).wait()

        src1 = chk_v.at[lax.rem(k + 1, 2)]

        @plsc.parallel_loop(0, NVEC, 1, unroll=2)
        def p1(i):
            idx16 = idx_v[0, pl.ds(i * NL, NL)]
            lu = plsc.bitcast(idx16 - SLOT_W, jnp.uint32)
            loc = plsc.bitcast(jnp.minimum(lu, swm1), jnp.int32)
            g = plsc.load_gather(src1, [zero16, loc])
            m = lu < swu
            plsc.store_scatter(
                ob_v.at[ob_s], [zero16, lane16 + i * NL], g, mask=m
            )

        chunk_copy(r, NSLOT_CH, k).wait()
        chunk_copy(r, 2, k).wait()

        @pl.when(k + 1 < RPW)
        def _():
            chunk_copy(r + 1, 0, k + 1).start()

        src2 = chk_v.at[lax.rem(k, 2)]

        @plsc.parallel_loop(0, NVEC, 1, unroll=2)
        def p2(i):
            idx16 = idx_v[0, pl.ds(i * NL, NL)]
            pos16 = lane16 + i * NL
            lu2 = plsc.bitcast(idx16 - 2 * SLOT_W, jnp.uint32)
            loc2 = plsc.bitcast(jnp.minimum(lu2, swm1), jnp.int32)
            g2 = plsc.load_gather(src2, [zero16, loc2])
            m2 = lu2 < swu
            plsc.store_scatter(ob_v.at[ob_s], [zero16, pos16], g2, mask=m2)
            lut = plsc.bitcast(idx16 - TAIL_B0, jnp.uint32)
            loct = plsc.bitcast(jnp.minimum(lut, twm1), jnp.int32)
            gt = plsc.load_gather(tail_v, [zero16, loct])
            mt = lut < twu
            plsc.store_scatter(ob_v.at[ob_s], [zero16, pos16], gt, mask=mt)

        pltpu.async_copy(ob_v.at[ob_s], out_hbm.at[pl.ds(r, 1)], wsem)
        return f

    chunk_copy(r0, 0, 0).start()
    lax.fori_loop(0, RPW, row_step, -1)
    for k in (RPW - 2, RPW - 1):
        pltpu.make_async_copy(
            ob_v.at[k % 2], out_hbm.at[pl.ds(r0 + k, 1)], wsem
        ).wait()


@jax.jit
def kernel(control_inputs, tables):
    tab_t = jnp.transpose(tables, (0, 2, 1))
    idx_t = jnp.transpose(control_inputs, (1, 0))

    mesh = plsc.VectorSubcoreMesh(core_axis_name="c", subcore_axis_name="s")
    out = pl.kernel(
        _body,
        mesh=mesh,
        out_type=jax.ShapeDtypeStruct((NROWS, BATCH), jnp.float32),
        scratch_types=[
            pltpu.VMEM((2, 1, SLOT_W), jnp.float32),
            pltpu.VMEM((1, TAIL_W), jnp.float32),
            pltpu.VMEM((1, BATCH), jnp.int32),
            pltpu.VMEM((2, 1, BATCH), jnp.float32),
            pltpu.SemaphoreType.DMA,
            pltpu.SemaphoreType.DMA,
        ],
        compiler_params=pltpu.CompilerParams(
            use_tc_tiling_on_sc=True, needs_layout_passes=False
        ),
    )(tab_t, idx_t)
    return jnp.transpose(out, (1, 0)).reshape(BATCH, NUM_FIELDS * EMBSIZE)


# speedup vs baseline: 1.2245x; 1.2245x over previous
import jax
import jax.numpy as jnp
from jax import lax
from jax.experimental import pallas as pl
from jax.experimental.pallas import tpu as pltpu
from jax.experimental.pallas import tpu_sc as plsc

NUM_FIELDS = 26
NUM_BUCKETS = 100000
EMBSIZE = 32
BATCH = 16384

_INFO = plsc.get_sparse_core_info()
NC, NS, NL = _INFO.num_cores, _INFO.num_subcores, _INFO.num_lanes
NW = NC * NS
NROWS = NUM_FIELDS * EMBSIZE
RPW = NROWS // NW
NVEC = BATCH // NL

SLOT_W = 25088
NSLOT_CH = 3
TAIL_B0 = NSLOT_CH * SLOT_W
TAIL_W = NUM_BUCKETS - TAIL_B0
NCH = NSLOT_CH + 1


def _body(tab_hbm, idx_hbm, out_hbm, chk_v, tail_v, idx_v, ob_v, gsem, wsem):
    wid = lax.axis_index("s") * NC + lax.axis_index("c")
    r0 = wid * RPW

    def chunk_copy(r, c, k):
        f = r // EMBSIZE
        e = r % EMBSIZE
        if c == NSLOT_CH:
            dst = tail_v
            b0, w = TAIL_B0, TAIL_W
        else:
            dst = chk_v.at[lax.rem(k + c, 2)]
            b0, w = c * SLOT_W, SLOT_W
        return pltpu.make_async_copy(
            tab_hbm.at[f, pl.ds(e, 1), pl.ds(b0, w)], dst, gsem
        )

    def row_step(k, f_prev):
        r = r0 + k
        f = r // EMBSIZE
        ob_s = lax.rem(k, 2)

        chunk_copy(r, 1, k).start()
        chunk_copy(r, NSLOT_CH, k).start()

        @pl.when(jnp.logical_or(k == 0, f != f_prev))
        def _():
            pltpu.sync_copy(idx_hbm.at[pl.ds(f, 1)], idx_v)

        @pl.when(k >= 2)
        def _():
            pltpu.make_async_copy(
                ob_v.at[ob_s], out_hbm.at[pl.ds(r - 2, 1)], wsem
            ).wait()

        zero16 = jnp.zeros((NL,), jnp.int32)
        lane16 = lax.iota(jnp.int32, NL)
        swu = jnp.uint32(SLOT_W)
        swm1 = jnp.uint32(SLOT_W - 1)
        twu = jnp.uint32(TAIL_W)
        twm1 = jnp.uint32(TAIL_W - 1)

        chunk_copy(r, 0, k).wait()

        src0 = chk_v.at[lax.rem(k, 2)]

        @plsc.parallel_loop(0, NVEC, 1, unroll=4)
        def p0(i):
            idx16 = idx_v[0, pl.ds(i * NL, NL)]
            lu = plsc.bitcast(idx16, jnp.uint32)
            loc = plsc.bitcast(jnp.minimum(lu, swm1), jnp.int32)
            g = plsc.load_gather(src0, [zero16, loc])
            ob_v.at[ob_s, 0][pl.ds(i * NL, NL)] = g

        chunk_copy(r, 2, k).start()
        chunk_copy(r, 1, k).wait()

        src1 = chk_v.at[lax.rem(k + 1, 2)]

        @plsc.parallel_loop(0, NVEC, 1, unroll=4)
        def p1(i):
            idx16 = idx_v[0, pl.ds(i * NL, NL)]
            lu = plsc.bitcast(idx16 - SLOT_W, jnp.uint32)
            loc = plsc.bitcast(jnp.minimum(lu, swm1), jnp.int32)
            g = plsc.load_gather(src1, [zero16, loc])
            m = lu < swu
            plsc.store_scatter(
                ob_v.at[ob_s], [zero16, lane16 + i * NL], g, mask=m
            )

        @pl.when(k + 1 < RPW)
        def _():
            chunk_copy(r + 1, 0, k + 1).start()

        chunk_copy(r, NSLOT_CH, k).wait()
        chunk_copy(r, 2, k).wait()

        src2 = chk_v.at[lax.rem(k, 2)]

        @plsc.parallel_loop(0, NVEC, 1, unroll=4)
        def p2(i):
            idx16 = idx_v[0, pl.ds(i * NL, NL)]
            pos16 = lane16 + i * NL
            lu2 = plsc.bitcast(idx16 - 2 * SLOT_W, jnp.uint32)
            loc2 = plsc.bitcast(jnp.minimum(lu2, swm1), jnp.int32)
            g2 = plsc.load_gather(src2, [zero16, loc2])
            m2 = lu2 < swu
            plsc.store_scatter(ob_v.at[ob_s], [zero16, pos16], g2, mask=m2)
            lut = plsc.bitcast(idx16 - TAIL_B0, jnp.uint32)
            loct = plsc.bitcast(jnp.minimum(lut, twm1), jnp.int32)
            gt = plsc.load_gather(tail_v, [zero16, loct])
            mt = lut < twu
            plsc.store_scatter(ob_v.at[ob_s], [zero16, pos16], gt, mask=mt)

        pltpu.async_copy(ob_v.at[ob_s], out_hbm.at[pl.ds(r, 1)], wsem)
        return f

    chunk_copy(r0, 0, 0).start()
    lax.fori_loop(0, RPW, row_step, -1)
    for k in (RPW - 2, RPW - 1):
        pltpu.make_async_copy(
            ob_v.at[k % 2], out_hbm.at[pl.ds(r0 + k, 1)], wsem
        ).wait()


@jax.jit
def kernel(control_inputs, tables):
    tab_t = jnp.transpose(tables, (0, 2, 1))
    idx_t = jnp.transpose(control_inputs, (1, 0))

    mesh = plsc.VectorSubcoreMesh(core_axis_name="c", subcore_axis_name="s")
    out = pl.kernel(
        _body,
        mesh=mesh,
        out_type=jax.ShapeDtypeStruct((NROWS, BATCH), jnp.float32),
        scratch_types=[
            pltpu.VMEM((2, 1, SLOT_W), jnp.float32),
            pltpu.VMEM((1, TAIL_W), jnp.float32),
            pltpu.VMEM((1, BATCH), jnp.int32),
            pltpu.VMEM((2, 1, BATCH), jnp.float32),
            pltpu.SemaphoreType.DMA,
            pltpu.SemaphoreType.DMA,
        ],
        compiler_params=pltpu.CompilerParams(
            use_tc_tiling_on_sc=True, needs_layout_passes=False
        ),
    )(tab_t, idx_t)
    return jnp.transpose(out, (1, 0)).reshape(BATCH, NUM_FIELDS * EMBSIZE)


# speedup vs baseline: 1.2275x; 1.0025x over previous
import jax
import jax.numpy as jnp
from jax import lax
from jax.experimental import pallas as pl
from jax.experimental.pallas import tpu as pltpu
from jax.experimental.pallas import tpu_sc as plsc

NUM_FIELDS = 26
NUM_BUCKETS = 100000
EMBSIZE = 32
BATCH = 16384

_INFO = plsc.get_sparse_core_info()
NC, NS, NL = _INFO.num_cores, _INFO.num_subcores, _INFO.num_lanes
NW = NC * NS
NROWS = NUM_FIELDS * EMBSIZE
RPW = NROWS // NW
NVEC = BATCH // NL

SLOT_W = 25088
NSLOT_CH = 3
TAIL_B0 = NSLOT_CH * SLOT_W
TAIL_W = NUM_BUCKETS - TAIL_B0
NCH = NSLOT_CH + 1


def _body(tab_hbm, idx_hbm, out_hbm, chk_v, tail_v, idx_v, ob_v, gsem, wsem):
    wid = lax.axis_index("s") * NC + lax.axis_index("c")
    r0 = wid * RPW

    def chunk_copy(r, c, k):
        f = r // EMBSIZE
        e = r % EMBSIZE
        if c == NSLOT_CH:
            dst = tail_v
            b0, w = TAIL_B0, TAIL_W
        else:
            dst = chk_v.at[lax.rem(k + c, 2)]
            b0, w = c * SLOT_W, SLOT_W
        return pltpu.make_async_copy(
            tab_hbm.at[f, pl.ds(e, 1), pl.ds(b0, w)], dst, gsem
        )

    def row_step(k, f_prev):
        r = r0 + k
        f = r // EMBSIZE
        ob_s = lax.rem(k, 2)

        @pl.when(jnp.logical_or(k == 0, f != f_prev))
        def _():
            pltpu.sync_copy(idx_hbm.at[pl.ds(f, 1)], idx_v)

        @pl.when(k >= 2)
        def _():
            pltpu.make_async_copy(
                ob_v.at[ob_s], out_hbm.at[pl.ds(r - 2, 1)], wsem
            ).wait()

        zero16 = jnp.zeros((NL,), jnp.int32)
        lane16 = lax.iota(jnp.int32, NL)
        swu = jnp.uint32(SLOT_W)
        swm1 = jnp.uint32(SLOT_W - 1)
        twu = jnp.uint32(TAIL_W)
        twm1 = jnp.uint32(TAIL_W - 1)

        chunk_copy(r, 1, k).start()
        chunk_copy(r, NSLOT_CH, k).start()
        chunk_copy(r, 0, k).wait()

        src0 = chk_v.at[lax.rem(k, 2)]

        @plsc.parallel_loop(0, NVEC, 1, unroll=4)
        def p0(i):
            idx16 = idx_v[0, pl.ds(i * NL, NL)]
            lu = plsc.bitcast(idx16, jnp.uint32)
            loc = plsc.bitcast(jnp.minimum(lu, swm1), jnp.int32)
            g = plsc.load_gather(src0, [zero16, loc])
            ob_v.at[ob_s, 0][pl.ds(i * NL, NL)] = g

        chunk_copy(r, 2, k).start()
        chunk_copy(r, 1, k).wait()

        src1 = chk_v.at[lax.rem(k + 1, 2)]

        @plsc.parallel_loop(0, NVEC, 1, unroll=4)
        def p1(i):
            idx16 = idx_v[0, pl.ds(i * NL, NL)]
            lu = plsc.bitcast(idx16 - SLOT_W, jnp.uint32)
            loc = plsc.bitcast(jnp.minimum(lu, swm1), jnp.int32)
            g = plsc.load_gather(src1, [zero16, loc])
            m = lu < swu
            plsc.store_scatter(
                ob_v.at[ob_s], [zero16, lane16 + i * NL], g, mask=m
            )

        @pl.when(k + 1 < RPW)
        def _():
            chunk_copy(r + 1, 0, k + 1).start()

        chunk_copy(r, NSLOT_CH, k).wait()
        chunk_copy(r, 2, k).wait()

        src2 = chk_v.at[lax.rem(k, 2)]

        @plsc.parallel_loop(0, NVEC, 1, unroll=4)
        def p2(i):
            idx16 = idx_v[0, pl.ds(i * NL, NL)]
            pos16 = lane16 + i * NL
            lu2 = plsc.bitcast(idx16 - 2 * SLOT_W, jnp.uint32)
            loc2 = plsc.bitcast(jnp.minimum(lu2, swm1), jnp.int32)
            g2 = plsc.load_gather(src2, [zero16, loc2])
            m2 = lu2 < swu
            plsc.store_scatter(ob_v.at[ob_s], [zero16, pos16], g2, mask=m2)
            lut = plsc.bitcast(idx16 - TAIL_B0, jnp.uint32)
            loct = plsc.bitcast(jnp.minimum(lut, twm1), jnp.int32)
            gt = plsc.load_gather(tail_v, [zero16, loct])
            mt = lut < twu
            plsc.store_scatter(ob_v.at[ob_s], [zero16, pos16], gt, mask=mt)

        pltpu.async_copy(ob_v.at[ob_s], out_hbm.at[pl.ds(r, 1)], wsem)
        return f

    chunk_copy(r0, 0, 0).start()
    lax.fori_loop(0, RPW, row_step, -1)
    for k in (RPW - 2, RPW - 1):
        pltpu.make_async_copy(
            ob_v.at[k % 2], out_hbm.at[pl.ds(r0 + k, 1)], wsem
        ).wait()


@jax.jit
def kernel(control_inputs, tables):
    tab_t = jnp.transpose(tables, (0, 2, 1))
    idx_t = jnp.transpose(control_inputs, (1, 0))

    mesh = plsc.VectorSubcoreMesh(core_axis_name="c", subcore_axis_name="s")
    out = pl.kernel(
        _body,
        mesh=mesh,
        out_type=jax.ShapeDtypeStruct((NROWS, BATCH), jnp.float32),
        scratch_types=[
            pltpu.VMEM((2, 1, SLOT_W), jnp.float32),
            pltpu.VMEM((1, TAIL_W), jnp.float32),
            pltpu.VMEM((1, BATCH), jnp.int32),
            pltpu.VMEM((2, 1, BATCH), jnp.float32),
            pltpu.SemaphoreType.DMA,
            pltpu.SemaphoreType.DMA,
        ],
        compiler_params=pltpu.CompilerParams(
            use_tc_tiling_on_sc=True, needs_layout_passes=False
        ),
    )(tab_t, idx_t)
    return jnp.transpose(out, (1, 0)).reshape(BATCH, NUM_FIELDS * EMBSIZE)
